# baseline (speedup 1.0000x reference)
import jax
import jax.numpy as jnp
from jax import lax
from jax.experimental import pallas as pl
from jax.experimental.pallas import tpu as pltpu, tpu_sc as plsc

NC = 2
NS = 16
NW = NC * NS
D = 64
CHUNK = 640


def _gather_body(ids_hbm, table_hbm, out_hbm, idx_v, rows0, rows1, sem0, sem1):
    b_per_w = ids_hbm.shape[1]
    n_chunks = b_per_w // CHUNK
    wid = lax.axis_index("s") * NC + lax.axis_index("c")
    base = wid * b_per_w

    pltpu.sync_copy(ids_hbm.at[wid], idx_v)

    rows = (rows0, rows1)
    sems = (sem0, sem1)

    def idx_slice(c):
        return idx_v.at[pl.ds(c * CHUNK, CHUNK)]

    for b in range(2):
        pltpu.async_copy(table_hbm.at[idx_slice(b)], rows[b], sems[b])

    @pl.loop(0, n_chunks - 2, step=2)
    def _(j):
        for b in range(2):
            c = j + b
            pltpu.make_async_copy(table_hbm.at[idx_slice(c)], rows[b], sems[b]).wait()
            pltpu.sync_copy(rows[b], out_hbm.at[pl.ds(base + c * CHUNK, CHUNK)])
            pltpu.async_copy(table_hbm.at[idx_slice(c + 2)], rows[b], sems[b])

    for b in range(2):
        c = n_chunks - 2 + b
        pltpu.make_async_copy(table_hbm.at[idx_slice(c)], rows[b], sems[b]).wait()
        pltpu.sync_copy(rows[b], out_hbm.at[pl.ds(base + c * CHUNK, CHUNK)])


def kernel(input_ids, attention_mask, table):
    del attention_mask
    batch, hist = input_ids.shape
    b_total = batch * hist
    assert b_total % (NW * CHUNK) == 0
    b_per_w = b_total // NW
    ids2 = input_ids.reshape(NW, b_per_w).astype(jnp.int32)

    run = pl.kernel(
        _gather_body,
        out_type=jax.ShapeDtypeStruct((b_total, D), jnp.float32),
        mesh=plsc.VectorSubcoreMesh(
            core_axis_name="c", subcore_axis_name="s",
            num_cores=NC, num_subcores=NS,
        ),
        compiler_params=pltpu.CompilerParams(use_tc_tiling_on_sc=False),
        scratch_types=[
            pltpu.VMEM((b_per_w,), jnp.int32),
            pltpu.VMEM((CHUNK, D), jnp.float32),
            pltpu.VMEM((CHUNK, D), jnp.float32),
            pltpu.SemaphoreType.DMA,
            pltpu.SemaphoreType.DMA,
        ],
    )
    out = run(ids2, table)
    return out.reshape(batch, hist, D)
